# baseline (speedup 1.0000x reference)
import jax
import jax.numpy as jnp
from jax import lax
from jax.experimental import pallas as pl
from jax.experimental.pallas import tpu as pltpu, tpu_sc as plsc

E = 320000
NB = E // 128
NC = 2
NS = 16
NW = NC * NS
WB = NB // NW
XW = NB % NW
CB = 20
CE = CB * 128
MAXB = WB + 1
NCHK = (MAXB + CB - 1) // CB
L = 16


def _edge_update(vj8, ea16, outp,
                 pidx, aidx,
                 vj0, vj1, ea0, ea1, ob0, ob1,
                 is0, is1, os0, os1):
    cid = lax.axis_index("c")
    sid = lax.axis_index("s")
    wid = sid * NC + cid
    b0 = wid * WB + jnp.minimum(wid, XW)
    nb = WB + (wid < XW).astype(jnp.int32)
    e0 = b0 * 128
    lanes = lax.iota(jnp.int32, L)
    c3 = jnp.full((L,), 3, jnp.int32)

    def build_pidx(i, carry):
        k = i * L + lanes
        pidx[pl.ds(i * L, L)] = jnp.minimum(e0 + k, E - 1) * 8
        return carry

    def build_aidx(i, carry):
        j = i * L + lanes
        row = (b0 + (j >> 3)) * 64 + (j & 7)
        aidx[pl.ds(i * L, L)] = jnp.minimum(row, E - 1)
        return carry

    bufs = [(vj0, ea0, ob0, is0, os0), (vj1, ea1, ob1, is1, os1)]
    starts = [jnp.minimum(c * CB, nb - CB) for c in range(NCHK)]

    def fire(c):
        vj_v, ea_v, _, isem, _ = bufs[c % 2]
        st = starts[c]
        pcp = pltpu.async_copy(
            vj8.at[pidx.at[pl.ds(st * 128, CE)]], vj_v, isem)
        acp = pltpu.async_copy(
            ea16.at[aidx.at[pl.ds(st * 8, CB * 8)]], ea_v, isem)
        return pcp, acp

    lax.fori_loop(0, CE // L, build_pidx, 0)
    lax.fori_loop(0, (CB * 8) // L, build_aidx, 0)
    inflight = fire(0)
    lax.fori_loop(CE // L, (MAXB * 128) // L, build_pidx, 0)
    lax.fori_loop((CB * 8) // L, (MAXB * 8 + L - 1) // L, build_aidx, 0)

    ocps = [None] * NCHK
    for c in range(NCHK):
        vj_v, ea_v, ob, _, osem = bufs[c % 2]
        pcp, acp = inflight
        if c + 1 < NCHK:
            inflight = fire(c + 1)
        pcp.wait()
        acp.wait()
        if c >= 2:
            ocps[c - 2].wait()

        def block(b, carry):
            for m in range(8):
                jj = b * 8 + m
                a = ea_v[jj, :]
                p = plsc.load_gather(vj_v, [b * 128 + m * L + lanes, c3])
                ob[pl.ds(b * 256 + m * L, L)] = a
                ob[pl.ds(b * 256 + 128 + m * L, L)] = a * p
            return carry

        lax.fori_loop(0, CB, block, 0)
        ocps[c] = pltpu.async_copy(
            ob, outp.at[pl.ds((b0 + starts[c]) * 256, CB * 256)], osem)

    ocps[NCHK - 2].wait()
    ocps[NCHK - 1].wait()


def kernel(vattr_i, vattr_j, edge_attr, g, batch):
    vj8 = vattr_j.reshape(E * 8, 16)
    ea16 = (
        edge_attr.reshape(NB, 128, 2, 8).transpose(2, 0, 3, 1).reshape(E, 16)
    )
    mesh = plsc.VectorSubcoreMesh(core_axis_name="c", subcore_axis_name="s")
    f = pl.kernel(
        _edge_update,
        out_type=jax.ShapeDtypeStruct((E * 2,), jnp.float32),
        mesh=mesh,
        compiler_params=pltpu.CompilerParams(
            use_tc_tiling_on_sc=False, needs_layout_passes=False
        ),
        scratch_types=[
            pltpu.VMEM((MAXB * 128,), jnp.int32),
            pltpu.VMEM((MAXB * 8 + 8,), jnp.int32),
            pltpu.VMEM((CE, 16), jnp.float32),
            pltpu.VMEM((CE, 16), jnp.float32),
            pltpu.VMEM((CB * 8, 16), jnp.float32),
            pltpu.VMEM((CB * 8, 16), jnp.float32),
            pltpu.VMEM((CB * 256,), jnp.float32),
            pltpu.VMEM((CB * 256,), jnp.float32),
            pltpu.SemaphoreType.DMA,
            pltpu.SemaphoreType.DMA,
            pltpu.SemaphoreType.DMA,
            pltpu.SemaphoreType.DMA,
        ],
    )
    outp = f(vj8, ea16)
    return outp.reshape(NB, 2, 128).transpose(0, 2, 1).reshape(E, 2)


# speedup vs baseline: 1.0105x; 1.0105x over previous
import jax
import jax.numpy as jnp
from jax import lax
from jax.experimental import pallas as pl
from jax.experimental.pallas import tpu as pltpu, tpu_sc as plsc

E = 320000
NB = E // 128
NC = 2
NS = 16
NW = NC * NS
WB = NB // NW
XW = NB % NW
CB = 16
CE = CB * 128
MAXB = WB + 1
NCHK = (MAXB + CB - 1) // CB
L = 16


def _edge_update(vj8, ea16, outp,
                 pidx, aidx,
                 vj0, vj1, ea0, ea1, ob0, ob1,
                 is0, is1, os0, os1):
    cid = lax.axis_index("c")
    sid = lax.axis_index("s")
    wid = sid * NC + cid
    b0 = wid * WB + jnp.minimum(wid, XW)
    nb = WB + (wid < XW).astype(jnp.int32)
    e0 = b0 * 128
    lanes = lax.iota(jnp.int32, L)
    c3 = jnp.full((L,), 3, jnp.int32)

    def build_pidx(i, carry):
        k = i * L + lanes
        pidx[pl.ds(i * L, L)] = jnp.minimum(e0 + k, E - 1) * 8
        return carry

    def build_aidx(i, carry):
        j = i * L + lanes
        row = (b0 + (j >> 3)) * 64 + (j & 7)
        aidx[pl.ds(i * L, L)] = jnp.minimum(row, E - 1)
        return carry

    bufs = [(vj0, ea0, ob0, is0, os0), (vj1, ea1, ob1, is1, os1)]
    starts = [jnp.minimum(c * CB, nb - CB) for c in range(NCHK)]

    def fire(c):
        vj_v, ea_v, _, isem, _ = bufs[c % 2]
        st = starts[c]
        pcp = pltpu.async_copy(
            vj8.at[pidx.at[pl.ds(st * 128, CE)]], vj_v, isem)
        acp = pltpu.async_copy(
            ea16.at[aidx.at[pl.ds(st * 8, CB * 8)]], ea_v, isem)
        return pcp, acp

    lax.fori_loop(0, CE // L, build_pidx, 0)
    lax.fori_loop(0, (CB * 8) // L, build_aidx, 0)
    inflight = fire(0)
    lax.fori_loop(CE // L, (MAXB * 128) // L, build_pidx, 0)
    lax.fori_loop((CB * 8) // L, (MAXB * 8 + L - 1) // L, build_aidx, 0)

    ocps = [None] * NCHK
    for c in range(NCHK):
        vj_v, ea_v, ob, _, osem = bufs[c % 2]
        pcp, acp = inflight
        if c + 1 < NCHK:
            inflight = fire(c + 1)
        pcp.wait()
        acp.wait()
        if c >= 2:
            ocps[c - 2].wait()

        def block(b, carry):
            for m in range(8):
                jj = b * 8 + m
                a = ea_v[jj, :]
                p = plsc.load_gather(vj_v, [b * 128 + m * L + lanes, c3])
                ob[pl.ds(b * 256 + m * L, L)] = a
                ob[pl.ds(b * 256 + 128 + m * L, L)] = a * p
            return carry

        lax.fori_loop(0, CB, block, 0)
        ocps[c] = pltpu.async_copy(
            ob, outp.at[pl.ds((b0 + starts[c]) * 256, CB * 256)], osem)

    ocps[NCHK - 2].wait()
    ocps[NCHK - 1].wait()


def kernel(vattr_i, vattr_j, edge_attr, g, batch):
    vj8 = vattr_j.reshape(E * 8, 16)
    ea16 = (
        edge_attr.reshape(NB, 128, 2, 8).transpose(2, 0, 3, 1).reshape(E, 16)
    )
    mesh = plsc.VectorSubcoreMesh(core_axis_name="c", subcore_axis_name="s")
    f = pl.kernel(
        _edge_update,
        out_type=jax.ShapeDtypeStruct((E * 2,), jnp.float32),
        mesh=mesh,
        compiler_params=pltpu.CompilerParams(
            use_tc_tiling_on_sc=False,
            needs_layout_passes=False,
            skip_device_barrier=True,
        ),
        scratch_types=[
            pltpu.VMEM((MAXB * 128,), jnp.int32),
            pltpu.VMEM((MAXB * 8 + 8,), jnp.int32),
            pltpu.VMEM((CE, 16), jnp.float32),
            pltpu.VMEM((CE, 16), jnp.float32),
            pltpu.VMEM((CB * 8, 16), jnp.float32),
            pltpu.VMEM((CB * 8, 16), jnp.float32),
            pltpu.VMEM((CB * 256,), jnp.float32),
            pltpu.VMEM((CB * 256,), jnp.float32),
            pltpu.SemaphoreType.DMA,
            pltpu.SemaphoreType.DMA,
            pltpu.SemaphoreType.DMA,
            pltpu.SemaphoreType.DMA,
        ],
    )
    outp = f(vj8, ea16)
    return outp.reshape(NB, 2, 128).transpose(0, 2, 1).reshape(E, 2)


# speedup vs baseline: 1.0148x; 1.0043x over previous
import jax
import jax.numpy as jnp
from jax import lax
from jax.experimental import pallas as pl
from jax.experimental.pallas import tpu as pltpu, tpu_sc as plsc

E = 320000
NB = E // 128
NC = 2
NS = 16
NW = NC * NS
WB = NB // NW
XW = NB % NW
CB = 16
CE = CB * 128
MAXB = WB + 1
NCHK = (MAXB + CB - 1) // CB
L = 16


def _edge_update(vj8, ea16, outp,
                 pidx, aidx,
                 vj0, vj1, ea0, ea1, ob0, ob1,
                 is0, is1, os0, os1):
    cid = lax.axis_index("c")
    sid = lax.axis_index("s")
    wid = sid * NC + cid
    b0 = wid * WB + jnp.minimum(wid, XW)
    nb = WB + (wid < XW).astype(jnp.int32)
    e0 = b0 * 128
    lanes = lax.iota(jnp.int32, L)
    c3 = jnp.full((L,), 3, jnp.int32)

    def build_pidx(i, carry):
        k = i * L + lanes
        pidx[pl.ds(i * L, L)] = jnp.minimum(e0 + k, E - 1) * 8
        return carry

    def build_aidx(i, carry):
        j = i * L + lanes
        row = (b0 + (j >> 3)) * 64 + (j & 7)
        aidx[pl.ds(i * L, L)] = jnp.minimum(row, E - 1)
        return carry

    bufs = [(vj0, ea0, ob0, is0, os0), (vj1, ea1, ob1, is1, os1)]
    starts = [jnp.minimum(c * CB, nb - CB) for c in range(NCHK)]

    def fire(c):
        vj_v, ea_v, _, isem, _ = bufs[c % 2]
        st = starts[c]
        h = CE // 2
        pcp0 = pltpu.async_copy(
            vj8.at[pidx.at[pl.ds(st * 128, h)]],
            vj_v.at[pl.ds(0, h), :], isem)
        pcp1 = pltpu.async_copy(
            vj8.at[pidx.at[pl.ds(st * 128 + h, h)]],
            vj_v.at[pl.ds(h, h), :], isem)
        acp = pltpu.async_copy(
            ea16.at[aidx.at[pl.ds(st * 8, CB * 8)]], ea_v, isem)
        return pcp0, pcp1, acp

    lax.fori_loop(0, CE // L, build_pidx, 0)
    lax.fori_loop(0, (CB * 8) // L, build_aidx, 0)
    inflight = fire(0)
    lax.fori_loop(CE // L, (MAXB * 128) // L, build_pidx, 0)
    lax.fori_loop((CB * 8) // L, (MAXB * 8 + L - 1) // L, build_aidx, 0)

    ocps = [None] * NCHK
    for c in range(NCHK):
        vj_v, ea_v, ob, _, osem = bufs[c % 2]
        pcp0, pcp1, acp = inflight
        if c + 1 < NCHK:
            inflight = fire(c + 1)
        pcp0.wait()
        pcp1.wait()
        acp.wait()
        if c >= 2:
            ocps[c - 2].wait()

        def block(b, carry):
            for m in range(8):
                jj = b * 8 + m
                a = ea_v[jj, :]
                p = plsc.load_gather(vj_v, [b * 128 + m * L + lanes, c3])
                ob[pl.ds(b * 256 + m * L, L)] = a
                ob[pl.ds(b * 256 + 128 + m * L, L)] = a * p
            return carry

        lax.fori_loop(0, CB, block, 0)
        ocps[c] = pltpu.async_copy(
            ob, outp.at[pl.ds((b0 + starts[c]) * 256, CB * 256)], osem)

    ocps[NCHK - 2].wait()
    ocps[NCHK - 1].wait()


def kernel(vattr_i, vattr_j, edge_attr, g, batch):
    vj8 = vattr_j.reshape(E * 8, 16)
    ea16 = (
        edge_attr.reshape(NB, 128, 2, 8).transpose(2, 0, 3, 1).reshape(E, 16)
    )
    mesh = plsc.VectorSubcoreMesh(core_axis_name="c", subcore_axis_name="s")
    f = pl.kernel(
        _edge_update,
        out_type=jax.ShapeDtypeStruct((E * 2,), jnp.float32),
        mesh=mesh,
        compiler_params=pltpu.CompilerParams(
            use_tc_tiling_on_sc=False,
            needs_layout_passes=False,
        ),
        scratch_types=[
            pltpu.VMEM((MAXB * 128,), jnp.int32),
            pltpu.VMEM((MAXB * 8 + 8,), jnp.int32),
            pltpu.VMEM((CE, 16), jnp.float32),
            pltpu.VMEM((CE, 16), jnp.float32),
            pltpu.VMEM((CB * 8, 16), jnp.float32),
            pltpu.VMEM((CB * 8, 16), jnp.float32),
            pltpu.VMEM((CB * 256,), jnp.float32),
            pltpu.VMEM((CB * 256,), jnp.float32),
            pltpu.SemaphoreType.DMA,
            pltpu.SemaphoreType.DMA,
            pltpu.SemaphoreType.DMA,
            pltpu.SemaphoreType.DMA,
        ],
    )
    outp = f(vj8, ea16)
    return outp.reshape(NB, 2, 128).transpose(0, 2, 1).reshape(E, 2)


# speedup vs baseline: 1.0610x; 1.0455x over previous
import jax
import jax.numpy as jnp
from jax import lax
from jax.experimental import pallas as pl
from jax.experimental.pallas import tpu as pltpu, tpu_sc as plsc

E = 320000
NB = E // 128
NC = 2
NS = 16
NW = NC * NS
WB = NB // NW
XW = NB % NW
CB = 16
CE = CB * 128
MAXB = WB + 1
NCHK = (MAXB + CB - 1) // CB
L = 16


def _edge_update(vj8, ea16, outp, pidx, aidx, vjb, eab, obb, isems, osems):
    cid = lax.axis_index("c")
    sid = lax.axis_index("s")
    wid = sid * NC + cid
    b0 = wid * WB + jnp.minimum(wid, XW)
    nb = WB + (wid < XW).astype(jnp.int32)
    e0 = b0 * 128
    lanes = lax.iota(jnp.int32, L)
    c3 = jnp.full((L,), 3, jnp.int32)

    def build_pidx(i, carry):
        k = i * L + lanes
        pidx[pl.ds(i * L, L)] = jnp.minimum(e0 + k, E - 1) * 8
        return carry

    def build_aidx(i, carry):
        j = i * L + lanes
        row = (b0 + (j >> 3)) * 64 + (j & 7)
        aidx[pl.ds(i * L, L)] = jnp.minimum(row, E - 1)
        return carry

    def start_of(c):
        return jnp.minimum(c * CB, nb - CB)

    def gather_copies(c, slot):
        st = start_of(c)
        pcp = pltpu.make_async_copy(
            vj8.at[pidx.at[pl.ds(st * 128, CE)]], vjb.at[slot], isems.at[slot])
        acp = pltpu.make_async_copy(
            ea16.at[aidx.at[pl.ds(st * 8, CB * 8)]], eab.at[slot],
            isems.at[slot])
        return pcp, acp

    def out_copy(c, slot):
        return pltpu.make_async_copy(
            obb.at[slot], outp.at[pl.ds((b0 + start_of(c)) * 256, CB * 256)],
            osems.at[slot])

    lax.fori_loop(0, CE // L, build_pidx, 0)
    lax.fori_loop(0, (CB * 8) // L, build_aidx, 0)
    for cp in gather_copies(0, 0):
        cp.start()
    lax.fori_loop(CE // L, (MAXB * 128) // L, build_pidx, 0)
    lax.fori_loop((CB * 8) // L, (MAXB * 8 + L - 1) // L, build_aidx, 0)

    def chunk(c, carry):
        slot = lax.rem(c, 2)
        nslot = lax.rem(c + 1, 2)

        @pl.when(c + 1 < NCHK)
        def _():
            for cp in gather_copies(c + 1, nslot):
                cp.start()

        for cp in gather_copies(c, slot):
            cp.wait()

        @pl.when(c >= 2)
        def _():
            out_copy(c - 2, slot).wait()

        ob = obb.at[slot]

        def block(b, carry2):
            for m in range(8):
                jj = b * 8 + m
                a = eab[slot, jj, :]
                p = plsc.load_gather(
                    vjb.at[slot], [b * 128 + m * L + lanes, c3])
                ob[pl.ds(b * 256 + m * L, L)] = a
                ob[pl.ds(b * 256 + 128 + m * L, L)] = a * p
            return carry2

        lax.fori_loop(0, CB, block, 0)
        out_copy(c, slot).start()
        return carry

    lax.fori_loop(0, NCHK, chunk, 0)
    out_copy(NCHK - 2, lax.rem(NCHK - 2, 2)).wait()
    out_copy(NCHK - 1, lax.rem(NCHK - 1, 2)).wait()


def kernel(vattr_i, vattr_j, edge_attr, g, batch):
    vj8 = vattr_j.reshape(E * 8, 16)
    ea16 = (
        edge_attr.reshape(NB, 128, 2, 8).transpose(2, 0, 3, 1).reshape(E, 16)
    )
    mesh = plsc.VectorSubcoreMesh(core_axis_name="c", subcore_axis_name="s")
    f = pl.kernel(
        _edge_update,
        out_type=jax.ShapeDtypeStruct((E * 2,), jnp.float32),
        mesh=mesh,
        compiler_params=pltpu.CompilerParams(
            use_tc_tiling_on_sc=False,
            needs_layout_passes=False,
        ),
        scratch_types=[
            pltpu.VMEM((MAXB * 128,), jnp.int32),
            pltpu.VMEM((MAXB * 8 + 8,), jnp.int32),
            pltpu.VMEM((2, CE, 16), jnp.float32),
            pltpu.VMEM((2, CB * 8, 16), jnp.float32),
            pltpu.VMEM((2, CB * 256), jnp.float32),
            pltpu.SemaphoreType.DMA((2,)),
            pltpu.SemaphoreType.DMA((2,)),
        ],
    )
    outp = f(vj8, ea16)
    return outp.reshape(NB, 2, 128).transpose(0, 2, 1).reshape(E, 2)


# speedup vs baseline: 1.0799x; 1.0178x over previous
import jax
import jax.numpy as jnp
from jax import lax
from jax.experimental import pallas as pl
from jax.experimental.pallas import tpu as pltpu, tpu_sc as plsc

E = 320000
NB = E // 128
NC = 2
NS = 16
NW = NC * NS
WB = NB // NW
XW = NB % NW
CB = 16
CE = CB * 128
MAXB = WB + 1
NCHK = (MAXB + CB - 1) // CB
L = 16


def _edge_update(vj8, ea16, outp, pidx, aidx, vjb, eab, obb, isems, osems):
    cid = lax.axis_index("c")
    sid = lax.axis_index("s")
    wid = sid * NC + cid
    b0 = wid * WB + jnp.minimum(wid, XW)
    nb = WB + (wid < XW).astype(jnp.int32)
    e0 = b0 * 128
    lanes = lax.iota(jnp.int32, L)
    c3 = jnp.full((L,), 3, jnp.int32)

    def build_pidx(i, carry):
        k = i * L + lanes
        pidx[pl.ds(i * L, L)] = jnp.minimum(e0 + k, E - 1) * 16
        return carry

    def build_aidx(i, carry):
        j = i * L + lanes
        row = (b0 + (j >> 3)) * 64 + (j & 7)
        aidx[pl.ds(i * L, L)] = jnp.minimum(row, E - 1)
        return carry

    def start_of(c):
        return jnp.minimum(c * CB, nb - CB)

    def gather_copies(c, slot):
        st = start_of(c)
        pcp = pltpu.make_async_copy(
            vj8.at[pidx.at[pl.ds(st * 128, CE)]], vjb.at[slot], isems.at[slot])
        acp = pltpu.make_async_copy(
            ea16.at[aidx.at[pl.ds(st * 8, CB * 8)]], eab.at[slot],
            isems.at[slot])
        return pcp, acp

    def out_copy(c, slot):
        return pltpu.make_async_copy(
            obb.at[slot], outp.at[pl.ds((b0 + start_of(c)) * 256, CB * 256)],
            osems.at[slot])

    lax.fori_loop(0, CE // L, build_pidx, 0)
    lax.fori_loop(0, (CB * 8) // L, build_aidx, 0)
    for cp in gather_copies(0, 0):
        cp.start()
    lax.fori_loop(CE // L, (MAXB * 128) // L, build_pidx, 0)
    lax.fori_loop((CB * 8) // L, (MAXB * 8 + L - 1) // L, build_aidx, 0)

    def chunk(c, carry):
        slot = lax.rem(c, 2)
        nslot = lax.rem(c + 1, 2)

        @pl.when(c + 1 < NCHK)
        def _():
            for cp in gather_copies(c + 1, nslot):
                cp.start()

        for cp in gather_copies(c, slot):
            cp.wait()

        @pl.when(c >= 2)
        def _():
            out_copy(c - 2, slot).wait()

        ob = obb.at[slot]

        def block(b, carry2):
            for m in range(8):
                jj = b * 8 + m
                a = eab[slot, jj, :]
                p = plsc.load_gather(
                    vjb.at[slot], [b * 128 + m * L + lanes, c3])
                ob[pl.ds(b * 256 + m * L, L)] = a
                ob[pl.ds(b * 256 + 128 + m * L, L)] = a * p
            return carry2

        lax.fori_loop(0, CB, block, 0)
        out_copy(c, slot).start()
        return carry

    lax.fori_loop(0, NCHK, chunk, 0)
    out_copy(NCHK - 2, lax.rem(NCHK - 2, 2)).wait()
    out_copy(NCHK - 1, lax.rem(NCHK - 1, 2)).wait()


def kernel(vattr_i, vattr_j, edge_attr, g, batch):
    vj8 = vattr_j.reshape(E * 16, 8)
    ea16 = (
        edge_attr.reshape(NB, 128, 2, 8).transpose(2, 0, 3, 1).reshape(E, 16)
    )
    mesh = plsc.VectorSubcoreMesh(core_axis_name="c", subcore_axis_name="s")
    f = pl.kernel(
        _edge_update,
        out_type=jax.ShapeDtypeStruct((E * 2,), jnp.float32),
        mesh=mesh,
        compiler_params=pltpu.CompilerParams(
            use_tc_tiling_on_sc=False,
            needs_layout_passes=False,
        ),
        scratch_types=[
            pltpu.VMEM((MAXB * 128,), jnp.int32),
            pltpu.VMEM((MAXB * 8 + 8,), jnp.int32),
            pltpu.VMEM((2, CE, 8), jnp.float32),
            pltpu.VMEM((2, CB * 8, 16), jnp.float32),
            pltpu.VMEM((2, CB * 256), jnp.float32),
            pltpu.SemaphoreType.DMA((2,)),
            pltpu.SemaphoreType.DMA((2,)),
        ],
    )
    outp = f(vj8, ea16)
    return outp.reshape(NB, 2, 128).transpose(0, 2, 1).reshape(E, 2)
